# baseline (speedup 1.0000x reference)
import functools

import jax
import jax.numpy as jnp
from jax import lax
from jax.experimental import pallas as pl
from jax.experimental.pallas import tpu as pltpu
from jax.experimental.pallas import tpu_sc as plsc

E = 8
K = 2
D = 1024
FF = 4096
T = 4096
A = T * K

BLK = 256
P = A + E * BLK
NB = P // BLK
S = 1
PS = P // S
NBS = NB // S

NC, NS = 2, 16
NW = NC * NS


def _router_body(x_ref, gw_ref, gb_ref, a_ref, p_ref):
    logits = jnp.dot(x_ref[...], gw_ref[...],
                     preferred_element_type=jnp.float32) + gb_ref[...]
    e_iota = lax.broadcasted_iota(jnp.int32, logits.shape, 1)
    m1 = jnp.max(logits, axis=1, keepdims=True)
    a1 = jnp.min(jnp.where(logits == m1, e_iota, E), axis=1, keepdims=True)
    masked = jnp.where(e_iota == a1, -jnp.inf, logits)
    m2 = jnp.max(masked, axis=1, keepdims=True)
    a2 = jnp.min(jnp.where(masked == m2, e_iota, E), axis=1, keepdims=True)
    a_ref[0:T] = a1
    a_ref[T:A] = a2
    p_ref[0:T] = 1.0 / (1.0 + jnp.exp(m2 - m1))
    p_ref[T:A] = 1.0 / (1.0 + jnp.exp(m1 - m2))


def _router(x, gate_w, gate_b):
    return pl.pallas_call(
        _router_body,
        out_shape=(jax.ShapeDtypeStruct((A, 1), jnp.int32),
                   jax.ShapeDtypeStruct((A, 1), jnp.float32)),
    )(x, gate_w, gate_b.reshape(1, E))


def _route_metadata(a_all):
    fe = a_all.reshape(-1)
    onehot = (fe[:, None] == jnp.arange(E)[None, :]).astype(jnp.int32)
    rank = jnp.cumsum(onehot, axis=0) - onehot
    rank_i = jnp.sum(rank * onehot, axis=1)
    counts = jnp.sum(onehot, axis=0)
    padded = ((counts + BLK - 1) // BLK) * BLK
    seg_end = jnp.cumsum(padded)
    seg_start = seg_end - padded
    dest = seg_start[fe] + rank_i
    blk_rows = jnp.arange(NB, dtype=jnp.int32) * BLK
    block_expert = jnp.sum(
        (seg_end[None, :] <= blk_rows[:, None]).astype(jnp.int32), axis=1)
    block_expert = jnp.minimum(block_expert, E - 1)
    nact = (seg_end[E - 1] + BLK - 1) // BLK
    d = dest.reshape(K, T)
    return (dest.reshape(NW, _S_N, _S_CH), block_expert,
            nact.reshape(1).astype(jnp.int32), d[0], d[1])


_A_PER_W = A // NW
_S_CH = 32
_S_N = _A_PER_W // _S_CH
_GBUF = 3
_LAG = 2
_NBUF = 3


@functools.cache
def _make_dispatch():
    return functools.partial(
        pl.kernel,
        out_type=(jax.ShapeDtypeStruct((P, D), jnp.float32),
                  jax.ShapeDtypeStruct((P,), jnp.float32)),
        mesh=plsc.VectorSubcoreMesh(core_axis_name="c", subcore_axis_name="s"),
        scratch_types=[
            pltpu.VMEM((_S_N, _S_CH), jnp.int32),
            pltpu.VMEM((_A_PER_W,), jnp.float32),
            *[pltpu.VMEM((_S_CH, D), jnp.float32) for _ in range(_GBUF)],
            *[pltpu.SemaphoreType.DMA for _ in range(3 * _GBUF)],
        ],
    )(_dispatch_body)


def _dispatch_body(x_hbm, dest_hbm, p_hbm, out_hbm, ws_hbm, idx_v, pv,
                   *bufs_and_sems):
    bufs = bufs_and_sems[:_GBUF]
    gsems = bufs_and_sems[_GBUF:2 * _GBUF]
    wsems = bufs_and_sems[2 * _GBUF:3 * _GBUF]
    psems = bufs_and_sems[3 * _GBUF:]
    wid = lax.axis_index("s") * NC + lax.axis_index("c")
    tb = (wid % (NW // K)) * _A_PER_W
    pltpu.sync_copy(dest_hbm.at[wid], idx_v)
    pltpu.sync_copy(p_hbm.at[pl.ds(wid * _A_PER_W, _A_PER_W)], pv)

    def start_g(i):
        b = i % _GBUF
        return pltpu.async_copy(
            x_hbm.at[pl.ds(tb + i * _S_CH, _S_CH)], bufs[b], gsems[b])

    gh = [None] * _S_N
    wh = [None] * _S_N
    ph = [None] * _S_N
    waited = [False] * _S_N
    head = min(_GBUF - _LAG, _S_N)
    for i in range(head):
        gh[i] = start_g(i)
    for i in range(_S_N):
        b = i % _GBUF
        j = i - _LAG + _GBUF
        if head <= j < _S_N:
            if i - _LAG >= 0:
                wh[i - _LAG].wait()
                ph[i - _LAG].wait()
                waited[i - _LAG] = True
            gh[j] = start_g(j)
        gh[i].wait()
        wh[i] = pltpu.async_copy(
            bufs[b], out_hbm.at[idx_v.at[i]], wsems[b])
        ph[i] = pltpu.async_copy(
            pv.at[pl.ds(i * _S_CH, _S_CH)], ws_hbm.at[idx_v.at[i]], psems[b])
    for i in range(_S_N):
        if not waited[i]:
            wh[i].wait()
            ph[i].wait()


FH = FF // 2


def _ffn_body1(be_ref, na_ref, x_ref, w1_ref, b1_ref, w2_ref, o_ref):
    @pl.when(pl.program_id(0) < na_ref[0])
    def _():
        h = jnp.dot(x_ref[...], w1_ref[0], preferred_element_type=jnp.float32)
        h = h + b1_ref[0]
        h = h / (1.0 + jnp.exp(-h))
        o_ref[...] = jnp.dot(h, w2_ref[0], preferred_element_type=jnp.float32)


def _ffn_body2(be_ref, na_ref, x_ref, w1_ref, b1_ref, w2_ref, b2_ref, ws_ref,
               y1_ref, o_ref):
    @pl.when(pl.program_id(0) < na_ref[0])
    def _():
        h = jnp.dot(x_ref[...], w1_ref[0], preferred_element_type=jnp.float32)
        h = h + b1_ref[0]
        h = h / (1.0 + jnp.exp(-h))
        y = jnp.dot(h, w2_ref[0], preferred_element_type=jnp.float32)
        o_ref[...] = (y1_ref[...] + y + b2_ref[0]) * ws_ref[...]


def _ffn_body2_acc(be_ref, na_ref, x_ref, w1_ref, b1_ref, w2_ref, b2_ref,
                   ws_ref, y1_ref, yin_ref, o_ref):
    del yin_ref
    _ffn_body2(be_ref, na_ref, x_ref, w1_ref, b1_ref, w2_ref, b2_ref, ws_ref,
               y1_ref, o_ref)


def _grouped_ffn_stage(s, block_expert_s, nact, x_s, w1, b1, w2, b2, ws_s,
                       y_prev):
    x_spec = pl.BlockSpec((BLK, D), lambda b, be, na: (b, 0))

    def w1_spec(f):
        return pl.BlockSpec((1, D, FH), lambda b, be, na: (be[b], 0, f))

    def b1_spec(f):
        return pl.BlockSpec((1, 1, FH), lambda b, be, na: (be[b], 0, f))

    def w2_spec(f):
        return pl.BlockSpec((1, FH, D), lambda b, be, na: (be[b], f, 0))

    grid1 = pltpu.PrefetchScalarGridSpec(
        num_scalar_prefetch=2,
        grid=(NBS,),
        in_specs=[x_spec, w1_spec(0), b1_spec(0), w2_spec(0)],
        out_specs=pl.BlockSpec((BLK, D), lambda b, be, na: (b, 0)),
    )
    y1 = pl.pallas_call(
        _ffn_body1,
        grid_spec=grid1,
        out_shape=jax.ShapeDtypeStruct((PS, D), jnp.float32),
    )(block_expert_s, nact, x_s, w1, b1, w2)

    o_spec = pl.BlockSpec((BLK, D), lambda b, be, na: (b + s * NBS, 0))
    in_specs = [x_spec, w1_spec(1), b1_spec(1), w2_spec(1),
                pl.BlockSpec((1, 1, D), lambda b, be, na: (be[b], 0, 0)),
                pl.BlockSpec((BLK, 1), lambda b, be, na: (b, 0)),
                pl.BlockSpec((BLK, D), lambda b, be, na: (b, 0))]
    args = [block_expert_s, nact, x_s, w1, b1, w2, b2, ws_s, y1]
    body = _ffn_body2
    aliases = {}
    if y_prev is not None:
        in_specs = in_specs + [o_spec]
        args = args + [y_prev]
        body = _ffn_body2_acc
        aliases = {9: 0}
    grid2 = pltpu.PrefetchScalarGridSpec(
        num_scalar_prefetch=2,
        grid=(NBS,),
        in_specs=in_specs,
        out_specs=o_spec,
    )
    return pl.pallas_call(
        body,
        grid_spec=grid2,
        out_shape=jax.ShapeDtypeStruct((P, D), jnp.float32),
        input_output_aliases=aliases,
    )(*args)


_C_CH = 16
_C_PER_W = T // NW


_C_N = _C_PER_W // _C_CH


@functools.cache
def _make_combine():
    return functools.partial(
        pl.kernel,
        out_type=jax.ShapeDtypeStruct((T, D), jnp.float32),
        mesh=plsc.VectorSubcoreMesh(core_axis_name="c", subcore_axis_name="s"),
        scratch_types=[
            pltpu.VMEM((_C_PER_W,), jnp.int32),
            pltpu.VMEM((_C_PER_W,), jnp.int32),
            *[pltpu.VMEM((_C_CH, D), jnp.float32) for _ in range(2 * _NBUF)],
            *[pltpu.SemaphoreType.DMA for _ in range(3 * _NBUF)],
        ],
    )(_combine_body)


def _combine_body(y_hbm, d0_hbm, d1_hbm, out_hbm,
                  i0_v, i1_v,
                  r00, r01, r02, r10, r11, r12,
                  g00, g01, g02, g10, g11, g12, ws0, ws1, ws2):
    wid = lax.axis_index("s") * NC + lax.axis_index("c")
    base = wid * _C_PER_W
    pltpu.sync_copy(d0_hbm.at[pl.ds(base, _C_PER_W)], i0_v)
    pltpu.sync_copy(d1_hbm.at[pl.ds(base, _C_PER_W)], i1_v)
    r0s = (r00, r01, r02)
    r1s = (r10, r11, r12)
    g0sems = (g00, g01, g02)
    g1sems = (g10, g11, g12)
    wsems = (ws0, ws1, ws2)

    def start_g(i):
        b = i % _NBUF
        sl = pl.ds(i * _C_CH, _C_CH)
        return (pltpu.async_copy(y_hbm.at[i0_v.at[sl]], r0s[b], g0sems[b]),
                pltpu.async_copy(y_hbm.at[i1_v.at[sl]], r1s[b], g1sems[b]))

    gh = [None] * _C_N
    wh = [None] * _C_N
    for i in range(min(_NBUF, _C_N)):
        gh[i] = start_g(i)
    for i in range(_C_N):
        b = i % _NBUF
        if i >= 1 and i + _NBUF - 1 < _C_N:
            wh[i - 1].wait()
            gh[i + _NBUF - 1] = start_g(i + _NBUF - 1)
        gh[i][0].wait()
        gh[i][1].wait()
        r0_v, r1_v = r0s[b], r1s[b]

        def col(c, carry2):
            sl = pl.ds(c * 16, 16)
            for r in range(_C_CH):
                r0_v[r, sl] = r0_v[r, sl] + r1_v[r, sl]
            return carry2

        lax.fori_loop(0, D // 16, col, 0)
        wh[i] = pltpu.async_copy(
            r0_v, out_hbm.at[pl.ds(base + i * _C_CH, _C_CH)], wsems[b])
    for i in range(max(0, _C_N - _NBUF), _C_N):
        wh[i].wait()


def kernel(inputs, gate_w, gate_b, w1, b1, w2, b2):
    a_all, p_all = _router(inputs, gate_w, gate_b)
    dest3, block_expert, nact, d0, d1 = _route_metadata(a_all)
    b1r = b1.reshape(E, 1, FF)
    b2r = b2.reshape(E, 1, D)
    x_sorted, ws = _make_dispatch()(inputs, dest3, p_all.reshape(-1))
    ws = ws.reshape(P, 1)
    y = None
    for s in range(S):
        y = _grouped_ffn_stage(
            s, block_expert[s * NBS:(s + 1) * NBS], nact,
            lax.slice_in_dim(x_sorted, s * PS, (s + 1) * PS),
            w1, b1r, w2, b2r,
            lax.slice_in_dim(ws, s * PS, (s + 1) * PS), y)
    return _make_combine()(y, d0, d1)


# speedup vs baseline: 1.0360x; 1.0360x over previous
import functools

import jax
import jax.numpy as jnp
from jax import lax
from jax.experimental import pallas as pl
from jax.experimental.pallas import tpu as pltpu
from jax.experimental.pallas import tpu_sc as plsc

E = 8
K = 2
D = 1024
FF = 4096
T = 4096
A = T * K

BLK = 512
P = A + E * BLK
NB = P // BLK
S = 1
PS = P // S
NBS = NB // S

NC, NS = 2, 16
NW = NC * NS


def _router_body(x_ref, gw_ref, gb_ref, a_ref, p_ref):
    logits = jnp.dot(x_ref[...], gw_ref[...],
                     preferred_element_type=jnp.float32) + gb_ref[...]
    e_iota = lax.broadcasted_iota(jnp.int32, logits.shape, 1)
    m1 = jnp.max(logits, axis=1, keepdims=True)
    a1 = jnp.min(jnp.where(logits == m1, e_iota, E), axis=1, keepdims=True)
    masked = jnp.where(e_iota == a1, -jnp.inf, logits)
    m2 = jnp.max(masked, axis=1, keepdims=True)
    a2 = jnp.min(jnp.where(masked == m2, e_iota, E), axis=1, keepdims=True)
    a_ref[0:T] = a1
    a_ref[T:A] = a2
    p_ref[0:T] = 1.0 / (1.0 + jnp.exp(m2 - m1))
    p_ref[T:A] = 1.0 / (1.0 + jnp.exp(m1 - m2))


def _router(x, gate_w, gate_b):
    return pl.pallas_call(
        _router_body,
        out_shape=(jax.ShapeDtypeStruct((A, 1), jnp.int32),
                   jax.ShapeDtypeStruct((A, 1), jnp.float32)),
    )(x, gate_w, gate_b.reshape(1, E))


def _route_metadata(a_all):
    fe = a_all.reshape(-1)
    onehot = (fe[:, None] == jnp.arange(E)[None, :]).astype(jnp.int32)
    rank = jnp.cumsum(onehot, axis=0) - onehot
    rank_i = jnp.sum(rank * onehot, axis=1)
    counts = jnp.sum(onehot, axis=0)
    padded = ((counts + BLK - 1) // BLK) * BLK
    seg_end = jnp.cumsum(padded)
    seg_start = seg_end - padded
    dest = seg_start[fe] + rank_i
    blk_rows = jnp.arange(NB, dtype=jnp.int32) * BLK
    block_expert = jnp.sum(
        (seg_end[None, :] <= blk_rows[:, None]).astype(jnp.int32), axis=1)
    block_expert = jnp.minimum(block_expert, E - 1)
    nact = (seg_end[E - 1] + BLK - 1) // BLK
    d = dest.reshape(K, T)
    return (dest.reshape(NW, _S_N, _S_CH), block_expert,
            nact.reshape(1).astype(jnp.int32), d[0], d[1])


_A_PER_W = A // NW
_S_CH = 32
_S_N = _A_PER_W // _S_CH
_GBUF = 3
_LAG = 2
_NBUF = 3


@functools.cache
def _make_dispatch():
    return functools.partial(
        pl.kernel,
        out_type=(jax.ShapeDtypeStruct((P, D), jnp.float32),
                  jax.ShapeDtypeStruct((P,), jnp.float32)),
        mesh=plsc.VectorSubcoreMesh(core_axis_name="c", subcore_axis_name="s"),
        scratch_types=[
            pltpu.VMEM((_S_N, _S_CH), jnp.int32),
            pltpu.VMEM((_A_PER_W,), jnp.float32),
            *[pltpu.VMEM((_S_CH, D), jnp.float32) for _ in range(_GBUF)],
            *[pltpu.SemaphoreType.DMA for _ in range(3 * _GBUF)],
        ],
    )(_dispatch_body)


def _dispatch_body(x_hbm, dest_hbm, p_hbm, out_hbm, ws_hbm, idx_v, pv,
                   *bufs_and_sems):
    bufs = bufs_and_sems[:_GBUF]
    gsems = bufs_and_sems[_GBUF:2 * _GBUF]
    wsems = bufs_and_sems[2 * _GBUF:3 * _GBUF]
    psems = bufs_and_sems[3 * _GBUF:]
    wid = lax.axis_index("s") * NC + lax.axis_index("c")
    tb = (wid % (NW // K)) * _A_PER_W
    pltpu.sync_copy(dest_hbm.at[wid], idx_v)
    pltpu.sync_copy(p_hbm.at[pl.ds(wid * _A_PER_W, _A_PER_W)], pv)

    def start_g(i):
        b = i % _GBUF
        return pltpu.async_copy(
            x_hbm.at[pl.ds(tb + i * _S_CH, _S_CH)], bufs[b], gsems[b])

    gh = [None] * _S_N
    wh = [None] * _S_N
    ph = [None] * _S_N
    waited = [False] * _S_N
    head = min(_GBUF - _LAG, _S_N)
    for i in range(head):
        gh[i] = start_g(i)
    for i in range(_S_N):
        b = i % _GBUF
        j = i - _LAG + _GBUF
        if head <= j < _S_N:
            if i - _LAG >= 0:
                wh[i - _LAG].wait()
                ph[i - _LAG].wait()
                waited[i - _LAG] = True
            gh[j] = start_g(j)
        gh[i].wait()
        wh[i] = pltpu.async_copy(
            bufs[b], out_hbm.at[idx_v.at[i]], wsems[b])
        ph[i] = pltpu.async_copy(
            pv.at[pl.ds(i * _S_CH, _S_CH)], ws_hbm.at[idx_v.at[i]], psems[b])
    for i in range(_S_N):
        if not waited[i]:
            wh[i].wait()
            ph[i].wait()


FH = FF // 2


def _ffn_body1(be_ref, na_ref, x_ref, w1_ref, b1_ref, w2_ref, o_ref):
    @pl.when(pl.program_id(0) < na_ref[0])
    def _():
        h = jnp.dot(x_ref[...], w1_ref[0], preferred_element_type=jnp.float32)
        h = h + b1_ref[0]
        h = h / (1.0 + jnp.exp(-h))
        o_ref[...] = jnp.dot(h, w2_ref[0], preferred_element_type=jnp.float32)


def _ffn_body2(be_ref, na_ref, x_ref, w1_ref, b1_ref, w2_ref, b2_ref, ws_ref,
               y1_ref, o_ref):
    @pl.when(pl.program_id(0) < na_ref[0])
    def _():
        h = jnp.dot(x_ref[...], w1_ref[0], preferred_element_type=jnp.float32)
        h = h + b1_ref[0]
        h = h / (1.0 + jnp.exp(-h))
        y = jnp.dot(h, w2_ref[0], preferred_element_type=jnp.float32)
        o_ref[...] = (y1_ref[...] + y + b2_ref[0]) * ws_ref[...]


def _ffn_body2_acc(be_ref, na_ref, x_ref, w1_ref, b1_ref, w2_ref, b2_ref,
                   ws_ref, y1_ref, yin_ref, o_ref):
    del yin_ref
    _ffn_body2(be_ref, na_ref, x_ref, w1_ref, b1_ref, w2_ref, b2_ref, ws_ref,
               y1_ref, o_ref)


def _grouped_ffn_stage(s, block_expert_s, nact, x_s, w1, b1, w2, b2, ws_s,
                       y_prev):
    x_spec = pl.BlockSpec((BLK, D), lambda b, be, na: (b, 0))

    def w1_spec(f):
        return pl.BlockSpec((1, D, FH), lambda b, be, na: (be[b], 0, f))

    def b1_spec(f):
        return pl.BlockSpec((1, 1, FH), lambda b, be, na: (be[b], 0, f))

    def w2_spec(f):
        return pl.BlockSpec((1, FH, D), lambda b, be, na: (be[b], f, 0))

    grid1 = pltpu.PrefetchScalarGridSpec(
        num_scalar_prefetch=2,
        grid=(NBS,),
        in_specs=[x_spec, w1_spec(0), b1_spec(0), w2_spec(0)],
        out_specs=pl.BlockSpec((BLK, D), lambda b, be, na: (b, 0)),
    )
    y1 = pl.pallas_call(
        _ffn_body1,
        grid_spec=grid1,
        out_shape=jax.ShapeDtypeStruct((PS, D), jnp.float32),
    )(block_expert_s, nact, x_s, w1, b1, w2)

    o_spec = pl.BlockSpec((BLK, D), lambda b, be, na: (b + s * NBS, 0))
    in_specs = [x_spec, w1_spec(1), b1_spec(1), w2_spec(1),
                pl.BlockSpec((1, 1, D), lambda b, be, na: (be[b], 0, 0)),
                pl.BlockSpec((BLK, 1), lambda b, be, na: (b, 0)),
                pl.BlockSpec((BLK, D), lambda b, be, na: (b, 0))]
    args = [block_expert_s, nact, x_s, w1, b1, w2, b2, ws_s, y1]
    body = _ffn_body2
    aliases = {}
    if y_prev is not None:
        in_specs = in_specs + [o_spec]
        args = args + [y_prev]
        body = _ffn_body2_acc
        aliases = {9: 0}
    grid2 = pltpu.PrefetchScalarGridSpec(
        num_scalar_prefetch=2,
        grid=(NBS,),
        in_specs=in_specs,
        out_specs=o_spec,
    )
    return pl.pallas_call(
        body,
        grid_spec=grid2,
        out_shape=jax.ShapeDtypeStruct((P, D), jnp.float32),
        input_output_aliases=aliases,
    )(*args)


_C_CH = 16
_C_PER_W = T // NW


_C_N = _C_PER_W // _C_CH


@functools.cache
def _make_combine():
    return functools.partial(
        pl.kernel,
        out_type=jax.ShapeDtypeStruct((T, D), jnp.float32),
        mesh=plsc.VectorSubcoreMesh(core_axis_name="c", subcore_axis_name="s"),
        scratch_types=[
            pltpu.VMEM((_C_PER_W,), jnp.int32),
            pltpu.VMEM((_C_PER_W,), jnp.int32),
            *[pltpu.VMEM((_C_CH, D), jnp.float32) for _ in range(2 * _NBUF)],
            *[pltpu.SemaphoreType.DMA for _ in range(3 * _NBUF)],
        ],
    )(_combine_body)


def _combine_body(y_hbm, d0_hbm, d1_hbm, out_hbm,
                  i0_v, i1_v,
                  r00, r01, r02, r10, r11, r12,
                  g00, g01, g02, g10, g11, g12, ws0, ws1, ws2):
    wid = lax.axis_index("s") * NC + lax.axis_index("c")
    base = wid * _C_PER_W
    pltpu.sync_copy(d0_hbm.at[pl.ds(base, _C_PER_W)], i0_v)
    pltpu.sync_copy(d1_hbm.at[pl.ds(base, _C_PER_W)], i1_v)
    r0s = (r00, r01, r02)
    r1s = (r10, r11, r12)
    g0sems = (g00, g01, g02)
    g1sems = (g10, g11, g12)
    wsems = (ws0, ws1, ws2)

    def start_g(i):
        b = i % _NBUF
        sl = pl.ds(i * _C_CH, _C_CH)
        return (pltpu.async_copy(y_hbm.at[i0_v.at[sl]], r0s[b], g0sems[b]),
                pltpu.async_copy(y_hbm.at[i1_v.at[sl]], r1s[b], g1sems[b]))

    gh = [None] * _C_N
    wh = [None] * _C_N
    for i in range(min(_NBUF, _C_N)):
        gh[i] = start_g(i)
    for i in range(_C_N):
        b = i % _NBUF
        if i >= 1 and i + _NBUF - 1 < _C_N:
            wh[i - 1].wait()
            gh[i + _NBUF - 1] = start_g(i + _NBUF - 1)
        gh[i][0].wait()
        gh[i][1].wait()
        r0_v, r1_v = r0s[b], r1s[b]

        def col(c, carry2):
            sl = pl.ds(c * 16, 16)
            for r in range(_C_CH):
                r0_v[r, sl] = r0_v[r, sl] + r1_v[r, sl]
            return carry2

        lax.fori_loop(0, D // 16, col, 0)
        wh[i] = pltpu.async_copy(
            r0_v, out_hbm.at[pl.ds(base + i * _C_CH, _C_CH)], wsems[b])
    for i in range(max(0, _C_N - _NBUF), _C_N):
        wh[i].wait()


def kernel(inputs, gate_w, gate_b, w1, b1, w2, b2):
    a_all, p_all = _router(inputs, gate_w, gate_b)
    dest3, block_expert, nact, d0, d1 = _route_metadata(a_all)
    b1r = b1.reshape(E, 1, FF)
    b2r = b2.reshape(E, 1, D)
    x_sorted, ws = _make_dispatch()(inputs, dest3, p_all.reshape(-1))
    ws = ws.reshape(P, 1)
    y = None
    for s in range(S):
        y = _grouped_ffn_stage(
            s, block_expert[s * NBS:(s + 1) * NBS], nact,
            lax.slice_in_dim(x_sorted, s * PS, (s + 1) * PS),
            w1, b1r, w2, b2r,
            lax.slice_in_dim(ws, s * PS, (s + 1) * PS), y)
    return _make_combine()(y, d0, d1)
